# baseline (speedup 1.0000x reference)
import functools

import jax
import jax.numpy as jnp
from jax import lax
from jax.experimental import pallas as pl
from jax.experimental.pallas import tpu as pltpu
from jax.experimental.pallas import tpu_sc as plsc

N = 10000
D = 128
E = 320000
COEFF = 0.1
NC = 2
NS = 16
EPT = E // (NC * NS)
C = 40
NCHUNK = EPT // C
NP = 10240
RPT = NP // NS


def _prepass_body(z_ref, nf_ref, zr_ref):
    zr_ref[...] = z_ref[...] * nf_ref[...]


def _prepass(z, nf):
    blk = 1000
    return pl.pallas_call(
        _prepass_body,
        grid=(N // blk,),
        in_specs=[
            pl.BlockSpec((blk, D), lambda i: (i, 0)),
            pl.BlockSpec((blk, 1), lambda i: (i, 0)),
        ],
        out_specs=pl.BlockSpec((blk, D), lambda i: (i, 0)),
        out_shape=jax.ShapeDtypeStruct((N, D), jnp.float32),
    )(z, nf)


def _postpass_body(z_ref, nf_ref, sa_ref, sb_ref, out_ref):
    s = sa_ref[...] + sb_ref[...]
    out_ref[...] = z_ref[...] - (2.0 * COEFF / N) * nf_ref[...] * s


def _postpass(z, nf, sa, sb):
    blk = 1000
    return pl.pallas_call(
        _postpass_body,
        grid=(N // blk,),
        in_specs=[
            pl.BlockSpec((blk, D), lambda i: (i, 0)),
            pl.BlockSpec((blk, 1), lambda i: (i, 0)),
            pl.BlockSpec((blk, D), lambda i: (i, 0)),
            pl.BlockSpec((blk, D), lambda i: (i, 0)),
        ],
        out_specs=pl.BlockSpec((blk, D), lambda i: (i, 0)),
        out_shape=jax.ShapeDtypeStruct((N, D), jnp.float32),
    )(z, nf, sa, sb)


def _sc_body(zr, rows, cols, zeros,
             sa, sb,
             idx_r, idx_c, buf_a, buf_b, acc, sem):
    c = lax.axis_index("c")
    s = lax.axis_index("s")

    r0 = s * RPT
    pltpu.sync_copy(zeros.at[pl.ds(r0, RPT)], acc.at[pl.ds(r0, RPT)])
    plsc.subcore_barrier()

    base = (c * NS + s) * EPT

    def chunk(k, _):
        off = base + k * C
        pltpu.sync_copy(rows.at[pl.ds(off, C)], idx_r)
        pltpu.sync_copy(cols.at[pl.ds(off, C)], idx_c)
        pltpu.async_copy(zr.at[idx_r], buf_a, sem).wait()
        pltpu.async_copy(zr.at[idx_c], buf_b, sem).wait()

        def diff(i, _):
            for f16 in range(D // 16):
                a = buf_a[i, pl.ds(f16 * 16, 16)]
                b = buf_b[i, pl.ds(f16 * 16, 16)]
                buf_a[i, pl.ds(f16 * 16, 16)] = a - b
                buf_b[i, pl.ds(f16 * 16, 16)] = b - a
            return 0

        lax.fori_loop(0, C, diff, 0)
        pltpu.sync_copy(buf_a, acc.at[idx_r], add=True)
        pltpu.sync_copy(buf_b, acc.at[idx_c], add=True)
        return 0

    lax.fori_loop(0, NCHUNK, chunk, 0)

    plsc.subcore_barrier()

    @pl.when(c == 0)
    def _():
        pltpu.sync_copy(acc.at[pl.ds(r0, RPT)], sa.at[pl.ds(r0, RPT)])

    @pl.when(c == 1)
    def _():
        pltpu.sync_copy(acc.at[pl.ds(r0, RPT)], sb.at[pl.ds(r0, RPT)])


_sc_kernel = functools.partial(
    pl.kernel,
    out_type=[
        jax.ShapeDtypeStruct((NP, D), jnp.float32),
        jax.ShapeDtypeStruct((NP, D), jnp.float32),
    ],
    mesh=plsc.VectorSubcoreMesh(
        core_axis_name="c", subcore_axis_name="s",
        num_cores=NC, num_subcores=NS,
    ),
    scratch_types=[
        pltpu.VMEM((C,), jnp.int32),
        pltpu.VMEM((C,), jnp.int32),
        pltpu.VMEM((C, D), jnp.float32),
        pltpu.VMEM((C, D), jnp.float32),
        pltpu.VMEM_SHARED((NP, D), jnp.float32),
        pltpu.SemaphoreType.DMA,
    ],
)(_sc_body)


@jax.jit
def kernel(z, x, edge_index, norm_factor):
    del x
    zr = _prepass(z, norm_factor)
    rows = edge_index[0]
    cols = edge_index[1]
    zeros = jnp.zeros((NP, D), jnp.float32)
    sa, sb = _sc_kernel(zr, rows, cols, zeros)
    return _postpass(z, norm_factor, sa, sb)


# speedup vs baseline: 1.0452x; 1.0452x over previous
import functools

import jax
import jax.numpy as jnp
from jax import lax
from jax.experimental import pallas as pl
from jax.experimental.pallas import tpu as pltpu
from jax.experimental.pallas import tpu_sc as plsc

N = 10000
D = 128
E = 320000
COEFF = 0.1
NC = 2
NS = 16
C = 128
NCHUNK = 79
EPT = NCHUNK * C
E2 = NC * NS * EPT
NP = 10240
RPT = NP // NS


def _prepass_body(z_ref, nf_ref, zr_ref):
    zr_ref[...] = z_ref[...] * nf_ref[...]


def _prepass(z, nf):
    blk = 1000
    return pl.pallas_call(
        _prepass_body,
        grid=(N // blk,),
        in_specs=[
            pl.BlockSpec((blk, D), lambda i: (i, 0)),
            pl.BlockSpec((blk, 1), lambda i: (i, 0)),
        ],
        out_specs=pl.BlockSpec((blk, D), lambda i: (i, 0)),
        out_shape=jax.ShapeDtypeStruct((N, D), jnp.float32),
    )(z, nf)


def _postpass_body(z_ref, nf_ref, sa_ref, sb_ref, out_ref):
    s = sa_ref[...] + sb_ref[...]
    out_ref[...] = z_ref[...] - (2.0 * COEFF / N) * nf_ref[...] * s


def _postpass(z, nf, sa, sb):
    blk = 1000
    return pl.pallas_call(
        _postpass_body,
        grid=(N // blk,),
        in_specs=[
            pl.BlockSpec((blk, D), lambda i: (i, 0)),
            pl.BlockSpec((blk, 1), lambda i: (i, 0)),
            pl.BlockSpec((blk, D), lambda i: (i, 0)),
            pl.BlockSpec((blk, D), lambda i: (i, 0)),
        ],
        out_specs=pl.BlockSpec((blk, D), lambda i: (i, 0)),
        out_shape=jax.ShapeDtypeStruct((N, D), jnp.float32),
    )(z, nf, sa, sb)


def _sc_body(zr, rows, cols, zeros,
             sa, sb,
             idx_r, idx_c, buf_a, buf_b, acc, sem):
    c = lax.axis_index("c")
    s = lax.axis_index("s")

    r0 = s * RPT
    pltpu.sync_copy(zeros.at[pl.ds(r0, RPT)], acc.at[pl.ds(r0, RPT)])
    plsc.subcore_barrier()

    base = (c * NS + s) * EPT

    def chunk(k, _):
        off = base + k * C
        pltpu.sync_copy(rows.at[pl.ds(off, C)], idx_r)
        pltpu.sync_copy(cols.at[pl.ds(off, C)], idx_c)
        pltpu.async_copy(zr.at[idx_r], buf_a, sem).wait()
        pltpu.async_copy(zr.at[idx_c], buf_b, sem).wait()

        def diff(i, _):
            for f16 in range(D // 16):
                a = buf_a[i, pl.ds(f16 * 16, 16)]
                b = buf_b[i, pl.ds(f16 * 16, 16)]
                buf_a[i, pl.ds(f16 * 16, 16)] = a - b
                buf_b[i, pl.ds(f16 * 16, 16)] = b - a
            return 0

        lax.fori_loop(0, C, diff, 0)
        pltpu.sync_copy(buf_a, acc.at[idx_r], add=True)
        pltpu.sync_copy(buf_b, acc.at[idx_c], add=True)
        return 0

    lax.fori_loop(0, NCHUNK, chunk, 0)

    plsc.subcore_barrier()

    @pl.when(c == 0)
    def _():
        pltpu.sync_copy(acc.at[pl.ds(r0, RPT)], sa.at[pl.ds(r0, RPT)])

    @pl.when(c == 1)
    def _():
        pltpu.sync_copy(acc.at[pl.ds(r0, RPT)], sb.at[pl.ds(r0, RPT)])


_sc_kernel = functools.partial(
    pl.kernel,
    out_type=[
        jax.ShapeDtypeStruct((NP, D), jnp.float32),
        jax.ShapeDtypeStruct((NP, D), jnp.float32),
    ],
    mesh=plsc.VectorSubcoreMesh(
        core_axis_name="c", subcore_axis_name="s",
        num_cores=NC, num_subcores=NS,
    ),
    scratch_types=[
        pltpu.VMEM((C,), jnp.int32),
        pltpu.VMEM((C,), jnp.int32),
        pltpu.VMEM((C, D), jnp.float32),
        pltpu.VMEM((C, D), jnp.float32),
        pltpu.VMEM_SHARED((NP, D), jnp.float32),
        pltpu.SemaphoreType.DMA,
    ],
)(_sc_body)


@jax.jit
def kernel(z, x, edge_index, norm_factor):
    del x
    zr = _prepass(z, norm_factor)
    pad = jnp.zeros((E2 - E,), jnp.int32)
    rows = jnp.concatenate([edge_index[0], pad])
    cols = jnp.concatenate([edge_index[1], pad])
    zeros = jnp.zeros((NP, D), jnp.float32)
    sa, sb = _sc_kernel(zr, rows, cols, zeros)
    return _postpass(z, norm_factor, sa, sb)


# speedup vs baseline: 1.5102x; 1.4449x over previous
import functools

import jax
import jax.numpy as jnp
from jax import lax
from jax.experimental import pallas as pl
from jax.experimental.pallas import tpu as pltpu
from jax.experimental.pallas import tpu_sc as plsc

N = 10000
D = 128
E = 320000
COEFF = 0.1
NC = 2
NS = 16
C = 80
NCHUNK = 126
EPT = NCHUNK * C
E2 = NC * NS * EPT
NP = 10240
RPT = NP // NS


def _prepass_body(z_ref, nf_ref, zr_ref):
    zr_ref[...] = z_ref[...] * nf_ref[...]


def _prepass(z, nf):
    blk = 1000
    return pl.pallas_call(
        _prepass_body,
        grid=(N // blk,),
        in_specs=[
            pl.BlockSpec((blk, D), lambda i: (i, 0)),
            pl.BlockSpec((blk, 1), lambda i: (i, 0)),
        ],
        out_specs=pl.BlockSpec((blk, D), lambda i: (i, 0)),
        out_shape=jax.ShapeDtypeStruct((N, D), jnp.float32),
    )(z, nf)


def _postpass_body(z_ref, nf_ref, sa_ref, sb_ref, out_ref):
    s = sa_ref[...] + sb_ref[...]
    out_ref[...] = z_ref[...] - (2.0 * COEFF / N) * nf_ref[...] * s


def _postpass(z, nf, sa, sb):
    blk = 1000
    return pl.pallas_call(
        _postpass_body,
        grid=(N // blk,),
        in_specs=[
            pl.BlockSpec((blk, D), lambda i: (i, 0)),
            pl.BlockSpec((blk, 1), lambda i: (i, 0)),
            pl.BlockSpec((blk, D), lambda i: (i, 0)),
            pl.BlockSpec((blk, D), lambda i: (i, 0)),
        ],
        out_specs=pl.BlockSpec((blk, D), lambda i: (i, 0)),
        out_shape=jax.ShapeDtypeStruct((N, D), jnp.float32),
    )(z, nf, sa, sb)


def _sc_body(zr, rows, cols, zeros,
             sa, sb,
             ir_a, ic_a, ir_b, ic_b, a0, b0, a1, b1, acc,
             g0, g1, s0, s1):
    c = lax.axis_index("c")
    s = lax.axis_index("s")

    r0 = s * RPT
    pltpu.sync_copy(zeros.at[pl.ds(r0, RPT)], acc.at[pl.ds(r0, RPT)])
    plsc.subcore_barrier()

    base = (c * NS + s) * EPT

    pltpu.sync_copy(rows.at[pl.ds(base, C)], ir_a)
    pltpu.sync_copy(cols.at[pl.ds(base, C)], ic_a)
    pltpu.async_copy(zr.at[ir_a], a0, g0)
    pltpu.async_copy(zr.at[ic_a], b0, g0)
    pltpu.sync_copy(rows.at[pl.ds(base + C, C)], ir_b)
    pltpu.sync_copy(cols.at[pl.ds(base + C, C)], ic_b)
    pltpu.async_copy(zr.at[ir_b], a1, g1)
    pltpu.async_copy(zr.at[ic_b], b1, g1)

    def diff(buf_a, buf_b):
        def body(i, _):
            for f16 in range(D // 16):
                sl = pl.ds(f16 * 16, 16)
                a = buf_a[i, sl]
                b = buf_b[i, sl]
                buf_a[i, sl] = a - b
                buf_b[i, sl] = b - a
            return 0

        lax.fori_loop(0, C, body, 0)

    @pl.loop(0, NCHUNK, step=2)
    def _(k):
        pltpu.make_async_copy(zr.at[ir_a], a0, g0).wait()
        pltpu.make_async_copy(zr.at[ic_a], b0, g0).wait()
        diff(a0, b0)
        sa0 = pltpu.async_copy(a0, acc.at[ir_a], s0, add=True)
        sb0 = pltpu.async_copy(b0, acc.at[ic_a], s0, add=True)

        pltpu.make_async_copy(zr.at[ir_b], a1, g1).wait()
        pltpu.make_async_copy(zr.at[ic_b], b1, g1).wait()
        diff(a1, b1)
        sa1 = pltpu.async_copy(a1, acc.at[ir_b], s1, add=True)
        sb1 = pltpu.async_copy(b1, acc.at[ic_b], s1, add=True)

        @pl.when(k + 2 < NCHUNK)
        def _():
            sa0.wait()
            sb0.wait()
            off = base + (k + 2) * C
            pltpu.sync_copy(rows.at[pl.ds(off, C)], ir_a)
            pltpu.sync_copy(cols.at[pl.ds(off, C)], ic_a)
            pltpu.async_copy(zr.at[ir_a], a0, g0)
            pltpu.async_copy(zr.at[ic_a], b0, g0)

        @pl.when(k + 3 < NCHUNK)
        def _():
            sa1.wait()
            sb1.wait()
            off = base + (k + 3) * C
            pltpu.sync_copy(rows.at[pl.ds(off, C)], ir_b)
            pltpu.sync_copy(cols.at[pl.ds(off, C)], ic_b)
            pltpu.async_copy(zr.at[ir_b], a1, g1)
            pltpu.async_copy(zr.at[ic_b], b1, g1)

    pltpu.make_async_copy(a0, acc.at[ir_a], s0).wait()
    pltpu.make_async_copy(b0, acc.at[ic_a], s0).wait()
    pltpu.make_async_copy(a1, acc.at[ir_b], s1).wait()
    pltpu.make_async_copy(b1, acc.at[ic_b], s1).wait()

    plsc.subcore_barrier()

    @pl.when(c == 0)
    def _():
        pltpu.sync_copy(acc.at[pl.ds(r0, RPT)], sa.at[pl.ds(r0, RPT)])

    @pl.when(c == 1)
    def _():
        pltpu.sync_copy(acc.at[pl.ds(r0, RPT)], sb.at[pl.ds(r0, RPT)])


_sc_kernel = functools.partial(
    pl.kernel,
    out_type=[
        jax.ShapeDtypeStruct((NP, D), jnp.float32),
        jax.ShapeDtypeStruct((NP, D), jnp.float32),
    ],
    mesh=plsc.VectorSubcoreMesh(
        core_axis_name="c", subcore_axis_name="s",
        num_cores=NC, num_subcores=NS,
    ),
    scratch_types=[
        pltpu.VMEM((C,), jnp.int32),
        pltpu.VMEM((C,), jnp.int32),
        pltpu.VMEM((C,), jnp.int32),
        pltpu.VMEM((C,), jnp.int32),
        pltpu.VMEM((C, D), jnp.float32),
        pltpu.VMEM((C, D), jnp.float32),
        pltpu.VMEM((C, D), jnp.float32),
        pltpu.VMEM((C, D), jnp.float32),
        pltpu.VMEM_SHARED((NP, D), jnp.float32),
        pltpu.SemaphoreType.DMA,
        pltpu.SemaphoreType.DMA,
        pltpu.SemaphoreType.DMA,
        pltpu.SemaphoreType.DMA,
    ],
)(_sc_body)


@jax.jit
def kernel(z, x, edge_index, norm_factor):
    del x
    zr = _prepass(z, norm_factor)
    pad = jnp.zeros((E2 - E,), jnp.int32)
    rows = jnp.concatenate([edge_index[0], pad])
    cols = jnp.concatenate([edge_index[1], pad])
    zeros = jnp.zeros((NP, D), jnp.float32)
    sa, sb = _sc_kernel(zr, rows, cols, zeros)
    return _postpass(z, norm_factor, sa, sb)


# speedup vs baseline: 2.5122x; 1.6635x over previous
import functools

import jax
import jax.numpy as jnp
from jax import lax
from jax.experimental import pallas as pl
from jax.experimental.pallas import tpu as pltpu
from jax.experimental.pallas import tpu_sc as plsc

N = 10000
D = 128
E = 320000
COEFF = 0.1
NC = 2
NS = 16
C = 80
NCHUNK = 126
EPT = NCHUNK * C
E2 = NC * NS * EPT
NP = 10240
RPT = NP // NS


def _prepass_body(z_ref, nf_ref, zr_ref):
    zr_ref[...] = z_ref[...] * nf_ref[...]


def _prepass(z, nf):
    blk = 1000
    return pl.pallas_call(
        _prepass_body,
        grid=(N // blk,),
        in_specs=[
            pl.BlockSpec((blk, D), lambda i: (i, 0)),
            pl.BlockSpec((blk, 1), lambda i: (i, 0)),
        ],
        out_specs=pl.BlockSpec((blk, D), lambda i: (i, 0)),
        out_shape=jax.ShapeDtypeStruct((N, D), jnp.float32),
    )(z, nf)


def _postpass_body(z_ref, nf_ref, sa_ref, sb_ref, out_ref):
    s = sa_ref[...] + sb_ref[...]
    out_ref[...] = z_ref[...] - (2.0 * COEFF / N) * nf_ref[...] * s


def _postpass(z, nf, sa, sb):
    blk = 1000
    return pl.pallas_call(
        _postpass_body,
        grid=(N // blk,),
        in_specs=[
            pl.BlockSpec((blk, D), lambda i: (i, 0)),
            pl.BlockSpec((blk, 1), lambda i: (i, 0)),
            pl.BlockSpec((blk, D), lambda i: (i, 0)),
            pl.BlockSpec((blk, D), lambda i: (i, 0)),
        ],
        out_specs=pl.BlockSpec((blk, D), lambda i: (i, 0)),
        out_shape=jax.ShapeDtypeStruct((N, D), jnp.float32),
    )(z, nf, sa, sb)


def _sc_body(zr, rows, cols, zeros,
             sa, sb,
             ir_a, ic_a, ir_b, ic_b, a0, b0, a1, b1, acc,
             g0, g1, s0, s1):
    c = lax.axis_index("c")
    s = lax.axis_index("s")

    r0 = s * RPT
    pltpu.sync_copy(zeros.at[pl.ds(r0, RPT)], acc.at[pl.ds(r0, RPT)])
    plsc.subcore_barrier()

    base = (c * NS + s) * EPT

    pltpu.sync_copy(rows.at[pl.ds(base, C)], ir_a)
    pltpu.sync_copy(cols.at[pl.ds(base, C)], ic_a)
    pltpu.async_copy(zr.at[ir_a], a0, g0)
    pltpu.async_copy(zr.at[ic_a], b0, g0)
    pltpu.sync_copy(rows.at[pl.ds(base + C, C)], ir_b)
    pltpu.sync_copy(cols.at[pl.ds(base + C, C)], ic_b)
    pltpu.async_copy(zr.at[ir_b], a1, g1)
    pltpu.async_copy(zr.at[ic_b], b1, g1)

    def diff(buf_a, buf_b):
        def body(i, _):
            for f16 in range(D // 16):
                sl = pl.ds(f16 * 16, 16)
                a = buf_a[i, sl]
                b = buf_b[i, sl]
                buf_a[i, sl] = a - b
                buf_b[i, sl] = b - a
            return 0

        lax.fori_loop(0, C, body, 0)

    @pl.loop(0, NCHUNK, step=2)
    def _(k):
        pltpu.make_async_copy(zr.at[ir_a], a0, g0).wait()
        pltpu.make_async_copy(zr.at[ic_a], b0, g0).wait()
        diff(a0, b0)
        sa0 = pltpu.async_copy(a0, acc.at[ir_a], s0, add=True)
        sb0 = pltpu.async_copy(b0, acc.at[ic_a], s0, add=True)

        pltpu.make_async_copy(zr.at[ir_b], a1, g1).wait()
        pltpu.make_async_copy(zr.at[ic_b], b1, g1).wait()
        diff(a1, b1)
        sa1 = pltpu.async_copy(a1, acc.at[ir_b], s1, add=True)
        sb1 = pltpu.async_copy(b1, acc.at[ic_b], s1, add=True)

        @pl.when(k + 2 < NCHUNK)
        def _():
            sa0.wait()
            sb0.wait()
            off = base + (k + 2) * C
            pltpu.sync_copy(rows.at[pl.ds(off, C)], ir_a)
            pltpu.sync_copy(cols.at[pl.ds(off, C)], ic_a)
            pltpu.async_copy(zr.at[ir_a], a0, g0)
            pltpu.async_copy(zr.at[ic_a], b0, g0)

        @pl.when(k + 3 < NCHUNK)
        def _():
            sa1.wait()
            sb1.wait()
            off = base + (k + 3) * C
            pltpu.sync_copy(rows.at[pl.ds(off, C)], ir_b)
            pltpu.sync_copy(cols.at[pl.ds(off, C)], ic_b)
            pltpu.async_copy(zr.at[ir_b], a1, g1)
            pltpu.async_copy(zr.at[ic_b], b1, g1)

    pltpu.make_async_copy(a0, acc.at[ir_a], s0).wait()
    pltpu.make_async_copy(b0, acc.at[ic_a], s0).wait()
    pltpu.make_async_copy(a1, acc.at[ir_b], s1).wait()
    pltpu.make_async_copy(b1, acc.at[ic_b], s1).wait()

    plsc.subcore_barrier()

    @pl.when(c == 0)
    def _():
        pltpu.sync_copy(acc.at[pl.ds(r0, RPT)], sa.at[pl.ds(r0, RPT)])

    @pl.when(c == 1)
    def _():
        pltpu.sync_copy(acc.at[pl.ds(r0, RPT)], sb.at[pl.ds(r0, RPT)])


_sc_kernel = functools.partial(
    pl.kernel,
    out_type=[
        jax.ShapeDtypeStruct((NP, D), jnp.float32),
        jax.ShapeDtypeStruct((NP, D), jnp.float32),
    ],
    mesh=plsc.VectorSubcoreMesh(
        core_axis_name="c", subcore_axis_name="s",
        num_cores=NC, num_subcores=NS,
    ),
    scratch_types=[
        pltpu.VMEM((C,), jnp.int32),
        pltpu.VMEM((C,), jnp.int32),
        pltpu.VMEM((C,), jnp.int32),
        pltpu.VMEM((C,), jnp.int32),
        pltpu.VMEM((C, D), jnp.float32),
        pltpu.VMEM((C, D), jnp.float32),
        pltpu.VMEM((C, D), jnp.float32),
        pltpu.VMEM((C, D), jnp.float32),
        pltpu.VMEM_SHARED((NP, D), jnp.float32),
        pltpu.SemaphoreType.DMA,
        pltpu.SemaphoreType.DMA,
        pltpu.SemaphoreType.DMA,
        pltpu.SemaphoreType.DMA,
    ],
)(_sc_body)


@jax.jit
def kernel(z, x, edge_index, norm_factor):
    del x
    zr = _prepass(z, norm_factor)
    pad = jnp.arange(E2 - E, dtype=jnp.int32) % N
    rows = jnp.concatenate([edge_index[0], pad])
    cols = jnp.concatenate([edge_index[1], pad])
    zeros = jnp.zeros((NP, D), jnp.float32)
    sa, sb = _sc_kernel(zr, rows, cols, zeros)
    return _postpass(z, norm_factor, sa, sb)
